# Initial kernel scaffold; baseline (speedup 1.0000x reference)
import jax
import jax.numpy as jnp
from jax.experimental import pallas as pl


def kernel(embed, edge_index_r0, edge_index_r1, edge_index_r2, h_bias1, weight2, h_bias2):
    raise NotImplementedError("write your pallas kernel here")


# speedup vs baseline: 2.8568x; 2.8568x over previous
import functools

import jax
import jax.numpy as jnp
from jax import lax
from jax.experimental import pallas as pl
from jax.experimental.pallas import tpu as pltpu
from jax.experimental.pallas import tpu_sc as plsc

N = 10000
H = 128
OUT = 64
R = 3
E = 320000

NC = 2
NS = 16
NW = NC * NS

CH = 128
CPT = 80
EPT = CPT * CH
EPAD = EPT * NW
RPT = 632
P = NS * RPT

BLK = 1264
GRID = P // BLK


def _make_edge_pass(width, ntab, with_deg):
    mesh = plsc.VectorSubcoreMesh(core_axis_name="c", subcore_axis_name="s")
    n_deg = R if with_deg else 0
    out_type = (
        [jax.ShapeDtypeStruct((NC, P, width), jnp.float32) for _ in range(R)]
        + [jax.ShapeDtypeStruct((NC, P), jnp.float32) for _ in range(n_deg)]
    )
    scratch = [
        pltpu.VMEM_SHARED((P, width), jnp.float32),
        pltpu.VMEM((CPT, CH), jnp.int32),
        pltpu.VMEM((CPT, CH), jnp.int32),
        pltpu.VMEM((CH, width), jnp.float32),
        pltpu.SemaphoreType.DMA,
    ]
    if with_deg:
        scratch.insert(1, pltpu.VMEM_SHARED((P,), jnp.float32))
        scratch.append(pltpu.VMEM((CH,), jnp.float32))

    @functools.partial(
        pl.kernel, out_type=out_type, scratch_types=scratch, mesh=mesh,
        name="edge_pass",
        compiler_params=pltpu.CompilerParams(use_tc_tiling_on_sc=False))
    def run(*refs):
        i = 0
        tabs = refs[i:i + ntab]; i += ntab
        srcs_hbm, dsts_hbm, z2_hbm, z1_hbm, ones_hbm = refs[i:i + 5]; i += 5
        out_acc = refs[i:i + R]; i += R
        out_deg = refs[i:i + n_deg]; i += n_deg
        acc = refs[i]; i += 1
        if with_deg:
            deg = refs[i]; i += 1
        srcb, dstb, rows, sem = refs[i:i + 4]; i += 4
        if with_deg:
            ones_v = refs[i]

        cid = lax.axis_index("c")
        sid = lax.axis_index("s")
        wid = sid * NC + cid
        rb = sid * RPT

        if with_deg:
            pltpu.sync_copy(ones_hbm, ones_v)

        for r in range(R):
            pltpu.sync_copy(z2_hbm, acc.at[pl.ds(rb, RPT)])
            if with_deg:
                @pl.when(sid == 0)
                def _zero_deg():
                    pltpu.sync_copy(z1_hbm, deg)
            plsc.subcore_barrier()

            pltpu.sync_copy(srcs_hbm.at[r].at[wid], srcb)
            pltpu.sync_copy(dsts_hbm.at[r].at[wid], dstb)
            tab = tabs[r % ntab]

            @pl.loop(0, CPT)
            def _chunk(j, tab=tab):
                cp = pltpu.async_copy(tab.at[srcb.at[j]], rows, sem)
                if with_deg:
                    pltpu.sync_copy(ones_v, deg.at[dstb.at[j]], add=True)
                cp.wait()
                pltpu.sync_copy(rows, acc.at[dstb.at[j]], add=True)

            plsc.subcore_barrier()

            pltpu.sync_copy(acc.at[pl.ds(rb, RPT)],
                            out_acc[r].at[cid].at[pl.ds(rb, RPT)])
            if with_deg:
                @pl.when(sid == 0)
                def _out_deg():
                    pltpu.sync_copy(deg, out_deg[r].at[cid])

    return run


@functools.lru_cache(maxsize=None)
def _edge_pass(width, ntab, with_deg):
    return _make_edge_pass(width, ntab, with_deg)


def _h1y_body(a0, a1, a2, dg0, dg1, dg2, b1, w2, y0, y1, y2):
    accs = (a0, a1, a2)
    dgs = (dg0, dg1, dg2)
    h = jnp.zeros((BLK, H), jnp.float32)
    for r in range(R):
        rec = 1.0 / jnp.maximum(dgs[r][0] + dgs[r][1], 1.0)
        h = h + (accs[r][0] + accs[r][1]) * rec
    h1 = jnp.maximum(h + b1[...][None, :], 0.0)
    for r, y in enumerate((y0, y1, y2)):
        y[...] = jnp.dot(h1, w2[r], preferred_element_type=jnp.float32)


def _tc_h1_y(acc3, deg3, b1, w2):
    acc_spec = pl.BlockSpec((NC, BLK, H), lambda i: (0, i, 0))
    deg_spec = pl.BlockSpec((NC, BLK, 1), lambda i: (0, i, 0))
    return pl.pallas_call(
        _h1y_body,
        grid=(GRID,),
        in_specs=[acc_spec] * 3 + [deg_spec] * 3
        + [pl.BlockSpec((H,), lambda i: (0,)),
           pl.BlockSpec((R, H, OUT), lambda i: (0, 0, 0))],
        out_specs=[pl.BlockSpec((BLK, OUT), lambda i: (i, 0))] * 3,
        out_shape=[jax.ShapeDtypeStruct((P, OUT), jnp.float32)] * 3,
    )(*acc3, *deg3, b1, w2)


def _out_body(a0, a1, a2, dg0, dg1, dg2, b2, o):
    accs = (a0, a1, a2)
    dgs = (dg0, dg1, dg2)
    h = jnp.zeros((BLK, OUT), jnp.float32)
    for r in range(R):
        rec = 1.0 / jnp.maximum(dgs[r][0] + dgs[r][1], 1.0)
        h = h + (accs[r][0] + accs[r][1]) * rec
    o[...] = h + b2[...][None, :]


def _tc_out(acc2, deg3, b2):
    acc_spec = pl.BlockSpec((NC, BLK, OUT), lambda i: (0, i, 0))
    deg_spec = pl.BlockSpec((NC, BLK, 1), lambda i: (0, i, 0))
    return pl.pallas_call(
        _out_body,
        grid=(GRID,),
        in_specs=[acc_spec] * 3 + [deg_spec] * 3
        + [pl.BlockSpec((OUT,), lambda i: (0,))],
        out_specs=pl.BlockSpec((BLK, OUT), lambda i: (i, 0)),
        out_shape=jax.ShapeDtypeStruct((P, OUT), jnp.float32),
    )(*acc2, *deg3, b2)


def kernel(embed, edge_index_r0, edge_index_r1, edge_index_r2,
           h_bias1, weight2, h_bias2):
    pad = EPAD - E
    srcs, dsts = [], []
    for e in (edge_index_r0, edge_index_r1, edge_index_r2):
        e = e.astype(jnp.int32)
        srcs.append(jnp.concatenate([e[0], jnp.zeros((pad,), jnp.int32)]))
        dsts.append(jnp.concatenate([e[1], jnp.full((pad,), N, jnp.int32)]))
    srcs = jnp.stack(srcs).reshape(R, NW, CPT, CH)
    dsts = jnp.stack(dsts).reshape(R, NW, CPT, CH)
    embed = embed.astype(jnp.float32)
    z2a = jnp.zeros((RPT, H), jnp.float32)
    z2b = jnp.zeros((RPT, OUT), jnp.float32)
    z1 = jnp.zeros((P,), jnp.float32)
    ones = jnp.ones((CH,), jnp.float32)

    res = _edge_pass(H, 1, True)(embed, srcs, dsts, z2a, z1, ones)
    acc1, deg3 = res[:R], res[R:]
    deg3 = [d.reshape(NC, P, 1) for d in deg3]

    ys = _tc_h1_y(acc1, deg3, h_bias1.astype(jnp.float32),
                  weight2.astype(jnp.float32))

    acc2 = _edge_pass(OUT, R, False)(ys[0], ys[1], ys[2], srcs, dsts,
                                     z2b, z1, ones)

    h2 = _tc_out(acc2, deg3, h_bias2.astype(jnp.float32))
    return h2[:N]


# speedup vs baseline: 3.4865x; 1.2204x over previous
import functools

import jax
import jax.numpy as jnp
from jax import lax
from jax.experimental import pallas as pl
from jax.experimental.pallas import tpu as pltpu
from jax.experimental.pallas import tpu_sc as plsc

N = 10000
H = 128
OUT = 64
R = 3
E = 320000

NC = 2
NS = 16
NW = NC * NS

CH = 128
S0 = 120
S1 = 40
NCHUNK = NS * (S0 + S1)
EPAD = NCHUNK * CH
IB = 20
PPB = IB // 2
RPT = 632
P = NS * RPT

BLK = 1264
GRID = P // BLK


def _make_edge_pass(width, ntab, with_deg):
    mesh = plsc.VectorSubcoreMesh(core_axis_name="c", subcore_axis_name="s")
    n_deg = R if with_deg else 0
    out_type = (
        [jax.ShapeDtypeStruct((NC, P, width), jnp.float32) for _ in range(R)]
        + [jax.ShapeDtypeStruct((NC, P), jnp.float32) for _ in range(n_deg)]
    )
    scratch = [
        pltpu.VMEM_SHARED((P, width), jnp.float32),
        pltpu.VMEM((IB, CH), jnp.int32),
        pltpu.VMEM((IB, CH), jnp.int32),
        pltpu.VMEM((CH, width), jnp.float32),
        pltpu.VMEM((CH, width), jnp.float32),
        pltpu.SemaphoreType.DMA,
        pltpu.SemaphoreType.DMA,
    ]
    if with_deg:
        scratch.insert(1, pltpu.VMEM_SHARED((P,), jnp.float32))
        scratch.append(pltpu.VMEM((CH,), jnp.float32))

    @functools.partial(
        pl.kernel, out_type=out_type, scratch_types=scratch, mesh=mesh,
        name="edge_pass",
        compiler_params=pltpu.CompilerParams(use_tc_tiling_on_sc=False))
    def run(*refs):
        i = 0
        tabs = refs[i:i + ntab]; i += ntab
        srcs_hbm, dsts_hbm, z2_hbm, z1_hbm, ones_hbm = refs[i:i + 5]; i += 5
        out_acc = refs[i:i + R]; i += R
        out_deg = refs[i:i + n_deg]; i += n_deg
        acc = refs[i]; i += 1
        if with_deg:
            deg = refs[i]; i += 1
        srcb, dstb, rows_a, rows_b, sem_a, sem_b = refs[i:i + 6]; i += 6
        if with_deg:
            ones_v = refs[i]

        cid = lax.axis_index("c")
        sid = lax.axis_index("s")
        rb = sid * RPT
        cbase = jnp.where(cid == 0, sid * S0, NS * S0 + sid * S1)
        npairs = jnp.where(cid == 0, S0 // 2, S1 // 2)

        if with_deg:
            pltpu.sync_copy(ones_hbm, ones_v)

        for r in range(R):
            pltpu.sync_copy(z2_hbm, acc.at[pl.ds(rb, RPT)])
            if with_deg:
                @pl.when(sid == 0)
                def _zero_deg():
                    pltpu.sync_copy(z1_hbm, deg)
            plsc.subcore_barrier()

            tab = tabs[r % ntab]

            @pl.loop(0, npairs)
            def _pair(p, tab=tab):
                @pl.when(p % PPB == 0)
                def _reload():
                    blk = pl.ds(cbase + 2 * p, IB)
                    pltpu.sync_copy(srcs_hbm.at[r].at[blk], srcb)
                    pltpu.sync_copy(dsts_hbm.at[r].at[blk], dstb)
                j0 = (2 * p) % IB
                j1 = j0 + 1
                cp0 = pltpu.async_copy(tab.at[srcb.at[j0]], rows_a, sem_a)
                cp1 = pltpu.async_copy(tab.at[srcb.at[j1]], rows_b, sem_b)
                if with_deg:
                    pltpu.sync_copy(ones_v, deg.at[dstb.at[j0]], add=True)
                    pltpu.sync_copy(ones_v, deg.at[dstb.at[j1]], add=True)
                cp0.wait()
                pltpu.sync_copy(rows_a, acc.at[dstb.at[j0]], add=True)
                cp1.wait()
                pltpu.sync_copy(rows_b, acc.at[dstb.at[j1]], add=True)

            plsc.subcore_barrier()

            pltpu.sync_copy(acc.at[pl.ds(rb, RPT)],
                            out_acc[r].at[cid].at[pl.ds(rb, RPT)])
            if with_deg:
                @pl.when(sid == 0)
                def _out_deg():
                    pltpu.sync_copy(deg, out_deg[r].at[cid])

    return run


@functools.lru_cache(maxsize=None)
def _edge_pass(width, ntab, with_deg):
    return _make_edge_pass(width, ntab, with_deg)


def _h1y_body(a0, a1, a2, dg0, dg1, dg2, b1, w2, y0, y1, y2):
    accs = (a0, a1, a2)
    dgs = (dg0, dg1, dg2)
    h = jnp.zeros((BLK, H), jnp.float32)
    for r in range(R):
        rec = 1.0 / jnp.maximum(dgs[r][0] + dgs[r][1], 1.0)
        h = h + (accs[r][0] + accs[r][1]) * rec
    h1 = jnp.maximum(h + b1[...][None, :], 0.0)
    for r, y in enumerate((y0, y1, y2)):
        y[...] = jnp.dot(h1, w2[r], preferred_element_type=jnp.float32)


def _tc_h1_y(acc3, deg3, b1, w2):
    acc_spec = pl.BlockSpec((NC, BLK, H), lambda i: (0, i, 0))
    deg_spec = pl.BlockSpec((NC, BLK, 1), lambda i: (0, i, 0))
    return pl.pallas_call(
        _h1y_body,
        grid=(GRID,),
        in_specs=[acc_spec] * 3 + [deg_spec] * 3
        + [pl.BlockSpec((H,), lambda i: (0,)),
           pl.BlockSpec((R, H, OUT), lambda i: (0, 0, 0))],
        out_specs=[pl.BlockSpec((BLK, OUT), lambda i: (i, 0))] * 3,
        out_shape=[jax.ShapeDtypeStruct((P, OUT), jnp.float32)] * 3,
    )(*acc3, *deg3, b1, w2)


def _out_body(a0, a1, a2, dg0, dg1, dg2, b2, o):
    accs = (a0, a1, a2)
    dgs = (dg0, dg1, dg2)
    h = jnp.zeros((BLK, OUT), jnp.float32)
    for r in range(R):
        rec = 1.0 / jnp.maximum(dgs[r][0] + dgs[r][1], 1.0)
        h = h + (accs[r][0] + accs[r][1]) * rec
    o[...] = h + b2[...][None, :]


def _tc_out(acc2, deg3, b2):
    acc_spec = pl.BlockSpec((NC, BLK, OUT), lambda i: (0, i, 0))
    deg_spec = pl.BlockSpec((NC, BLK, 1), lambda i: (0, i, 0))
    return pl.pallas_call(
        _out_body,
        grid=(GRID,),
        in_specs=[acc_spec] * 3 + [deg_spec] * 3
        + [pl.BlockSpec((OUT,), lambda i: (0,))],
        out_specs=pl.BlockSpec((BLK, OUT), lambda i: (i, 0)),
        out_shape=jax.ShapeDtypeStruct((P, OUT), jnp.float32),
    )(*acc2, *deg3, b2)


def kernel(embed, edge_index_r0, edge_index_r1, edge_index_r2,
           h_bias1, weight2, h_bias2):
    pad = EPAD - E
    srcs, dsts = [], []
    for e in (edge_index_r0, edge_index_r1, edge_index_r2):
        e = e.astype(jnp.int32)
        srcs.append(jnp.concatenate([e[0], jnp.zeros((pad,), jnp.int32)]))
        dsts.append(jnp.concatenate([e[1], jnp.full((pad,), N, jnp.int32)]))
    srcs = jnp.stack(srcs).reshape(R, NCHUNK, CH)
    dsts = jnp.stack(dsts).reshape(R, NCHUNK, CH)
    embed = embed.astype(jnp.float32)
    z2a = jnp.zeros((RPT, H), jnp.float32)
    z2b = jnp.zeros((RPT, OUT), jnp.float32)
    z1 = jnp.zeros((P,), jnp.float32)
    ones = jnp.ones((CH,), jnp.float32)

    res = _edge_pass(H, 1, True)(embed, srcs, dsts, z2a, z1, ones)
    acc1, deg3 = res[:R], res[R:]
    deg3 = [d.reshape(NC, P, 1) for d in deg3]

    ys = _tc_h1_y(acc1, deg3, h_bias1.astype(jnp.float32),
                  weight2.astype(jnp.float32))

    acc2 = _edge_pass(OUT, R, False)(ys[0], ys[1], ys[2], srcs, dsts,
                                     z2b, z1, ones)

    h2 = _tc_out(acc2, deg3, h_bias2.astype(jnp.float32))
    return h2[:N]
